# baseline (speedup 1.0000x reference)
import dataclasses
import functools

import jax
import jax.numpy as jnp
from jax import lax
from jax.experimental import pallas as pl
from jax.experimental.pallas import tpu as pltpu
from jax.experimental.pallas import tpu_sc as plsc

B, M, C, H, W = 8, 2048, 192, 224, 224
HW = H * W
CP = 256

NC, NS, L = 2, 16, 16
NW = NC * NS
KPW = (B * M) // NW
CK = 128
NCH = KPW // CK
NBUF = 2

TW = 1792

_mesh = plsc.VectorSubcoreMesh(core_axis_name="c", subcore_axis_name="s")

_cp = pltpu.CompilerParams()
if "needs_layout_passes" in pltpu.CompilerParams.__dataclass_fields__:
    _cp = dataclasses.replace(_cp, needs_layout_passes=False)


def _transpose_kernel(seg_ref, out_ref):
    out_ref[0, :, :C] = seg_ref[0].T


_transpose = pl.pallas_call(
    _transpose_kernel,
    grid=(B, HW // TW),
    in_specs=[pl.BlockSpec((1, C, TW), lambda b, t: (b, 0, t))],
    out_specs=pl.BlockSpec((1, TW, CP), lambda b, t: (b, t, 0)),
    out_shape=jax.ShapeDtypeStruct((B, HW, CP), jnp.float32),
)


@functools.partial(
    pl.kernel,
    mesh=_mesh,
    compiler_params=_cp,
    out_type=jax.ShapeDtypeStruct((B * M, CP), jnp.float32),
    scratch_types=[
        pltpu.VMEM((KPW * 2,), jnp.float32),
        pltpu.VMEM((KPW,), jnp.int32),
        pltpu.VMEM((NBUF, CK, CP), jnp.float32),
        pltpu.SemaphoreType.DMA,
    ],
)
def _gather_rows(kpts_hbm, table_hbm, out_hbm, kv, iv, rv, sem):
    wid = lax.axis_index("s") * NC + lax.axis_index("c")
    kp0 = wid * KPW
    b = kp0 // M
    rbase = b * HW

    lanes = lax.iota(jnp.int32, L)

    pltpu.sync_copy(kpts_hbm.at[pl.ds(kp0 * 2, KPW * 2)], kv)

    for j in range(KPW // L):
        ev = lanes * 2 + (j * 2 * L)
        xf = plsc.load_gather(kv, [ev]) * float(H)
        yf = plsc.load_gather(kv, [ev + 1]) * float(W)
        xi = xf.astype(jnp.int32)
        xi = jnp.where(xi.astype(jnp.float32) > xf, xi - 1, xi)
        yi = yf.astype(jnp.int32)
        yi = jnp.where(yi.astype(jnp.float32) > yf, yi - 1, yi)
        p = jnp.minimum(jnp.maximum(xi * W + yi, 0), HW - 1)
        iv[pl.ds(j * L, L)] = p + rbase

    def _fire(t):
        return pltpu.async_copy(
            table_hbm.at[iv.at[pl.ds(t * CK, CK)]],
            rv.at[t % NBUF],
            sem,
        )

    cps = [_fire(t) for t in range(NBUF)]
    for t in range(NCH):
        cps[t % NBUF].wait()
        pltpu.sync_copy(rv.at[t % NBUF], out_hbm.at[pl.ds(kp0 + t * CK, CK)])
        if t + NBUF < NCH:
            cps[t % NBUF] = _fire(t + NBUF)


def kernel(original_kpts, segment):
    table = _transpose(segment.reshape(B, C, HW)).reshape(B * HW, CP)
    out = _gather_rows(original_kpts.reshape(-1), table)
    return out[:, :C].reshape(B, M, C)


# speedup vs baseline: 1.0031x; 1.0031x over previous
import dataclasses
import functools

import jax
import jax.numpy as jnp
from jax import lax
from jax.experimental import pallas as pl
from jax.experimental.pallas import tpu as pltpu
from jax.experimental.pallas import tpu_sc as plsc

B, M, C, H, W = 8, 2048, 192, 224, 224
HW = H * W

NC, NS, L = 2, 16, 16
NW = NC * NS
KPW = (B * M) // NW
CK = 128
NG = CK // L
SLICE = 2048
NSL = (CK * C) // SLICE
FIRE = 4
CU = 8

_mesh = plsc.VectorSubcoreMesh(core_axis_name="c", subcore_axis_name="s")

_cp = pltpu.CompilerParams()
if "needs_layout_passes" in pltpu.CompilerParams.__dataclass_fields__:
    _cp = dataclasses.replace(_cp, needs_layout_passes=False)


@functools.partial(
    pl.kernel,
    mesh=_mesh,
    compiler_params=_cp,
    out_type=jax.ShapeDtypeStruct((B * M * C,), jnp.float32),
    scratch_types=[
        pltpu.VMEM((CK * 2,), jnp.float32),
        pltpu.VMEM((CK,), jnp.int32),
        pltpu.VMEM((CK * C,), jnp.int32),
        pltpu.VMEM((CK * C,), jnp.float32),
        pltpu.SemaphoreType.DMA,
    ],
)
def _gather_si(kpts_hbm, seg_hbm, out_hbm, kv, bv, iv, gv, sem):
    wid = lax.axis_index("s") * NC + lax.axis_index("c")
    kp0 = wid * KPW
    b = kp0 // M
    bbase = b * (C * HW)

    lanes = lax.iota(jnp.int32, L)

    @pl.loop(0, KPW // CK)
    def _chunk(t):
        row0 = kp0 + t * CK
        pltpu.sync_copy(kpts_hbm.at[pl.ds(row0 * 2, CK * 2)], kv)

        for j in range(NG):
            ev = lanes * 2 + (j * 2 * L)
            xf = plsc.load_gather(kv, [ev]) * float(H)
            yf = plsc.load_gather(kv, [ev + 1]) * float(W)
            xi = xf.astype(jnp.int32)
            xi = jnp.where(xi.astype(jnp.float32) > xf, xi - 1, xi)
            yi = yf.astype(jnp.int32)
            yi = jnp.where(yi.astype(jnp.float32) > yf, yi - 1, yi)
            p = jnp.minimum(jnp.maximum(xi * W + yi, 0), HW - 1)
            bv[pl.ds(j * L, L)] = p + bbase

        for g in range(NG):
            base16 = bv[pl.ds(g * L, L)]
            gbase = g * C * L

            @pl.loop(0, C // CU)
            def _expand(cq):
                pos = gbase + cq * (CU * L)
                coff = cq * (CU * HW)
                for u in range(CU):
                    iv[pl.ds(pos + u * L, L)] = base16 + (coff + u * HW)

        for s in range(0, NSL, FIRE):
            cps = [
                pltpu.async_copy(
                    seg_hbm.at[iv.at[pl.ds((s + r) * SLICE, SLICE)]],
                    gv.at[pl.ds((s + r) * SLICE, SLICE)],
                    sem,
                )
                for r in range(FIRE)
            ]
            for cp in cps:
                cp.wait()

        pltpu.sync_copy(gv, out_hbm.at[pl.ds(row0 * C, CK * C)])


def kernel(original_kpts, segment):
    out = _gather_si(original_kpts.reshape(-1), segment.reshape(-1))
    out = out.reshape(B * M // L, C, L)
    return jnp.transpose(out, (0, 2, 1)).reshape(B, M, C)


# speedup vs baseline: 1.7308x; 1.7254x over previous
import dataclasses
import functools

import jax
import jax.numpy as jnp
from jax import lax
from jax.experimental import pallas as pl
from jax.experimental.pallas import tpu as pltpu
from jax.experimental.pallas import tpu_sc as plsc

B, M, C, H, W = 8, 2048, 192, 224, 224
HW = H * W
CP = 256

NC, NS, L = 2, 16, 16
NW = NC * NS
KPW = (B * M) // NW
CK = 128
NCH = KPW // CK
NBUF = 2

HB = 8

_mesh = plsc.VectorSubcoreMesh(core_axis_name="c", subcore_axis_name="s")

_cp = pltpu.CompilerParams()
if "needs_layout_passes" in pltpu.CompilerParams.__dataclass_fields__:
    _cp = dataclasses.replace(_cp, needs_layout_passes=False)


def _transpose_kernel(seg_ref, out_ref):
    x = seg_ref[0]
    for h in range(HB):
        out_ref[0, h, :, :C] = x[:, h, :].T


_transpose = pl.pallas_call(
    _transpose_kernel,
    grid=(B, H // HB),
    in_specs=[pl.BlockSpec((1, C, HB, W), lambda b, t: (b, 0, t, 0))],
    out_specs=pl.BlockSpec((1, HB, W, CP), lambda b, t: (b, t, 0, 0)),
    out_shape=jax.ShapeDtypeStruct((B, H, W, CP), jnp.float32),
)


@functools.partial(
    pl.kernel,
    mesh=_mesh,
    compiler_params=_cp,
    out_type=jax.ShapeDtypeStruct((B * M, CP), jnp.float32),
    scratch_types=[
        pltpu.VMEM((KPW * 2,), jnp.float32),
        pltpu.VMEM((KPW,), jnp.int32),
        pltpu.VMEM((NBUF, CK, CP), jnp.float32),
        pltpu.SemaphoreType.DMA,
    ],
)
def _gather_rows(kpts_hbm, table_hbm, out_hbm, kv, iv, rv, sem):
    wid = lax.axis_index("s") * NC + lax.axis_index("c")
    kp0 = wid * KPW
    b = kp0 // M
    rbase = b * HW

    lanes = lax.iota(jnp.int32, L)

    pltpu.sync_copy(kpts_hbm.at[pl.ds(kp0 * 2, KPW * 2)], kv)

    for j in range(KPW // L):
        ev = lanes * 2 + (j * 2 * L)
        xf = plsc.load_gather(kv, [ev]) * float(H)
        yf = plsc.load_gather(kv, [ev + 1]) * float(W)
        xi = xf.astype(jnp.int32)
        xi = jnp.where(xi.astype(jnp.float32) > xf, xi - 1, xi)
        yi = yf.astype(jnp.int32)
        yi = jnp.where(yi.astype(jnp.float32) > yf, yi - 1, yi)
        p = jnp.minimum(jnp.maximum(xi * W + yi, 0), HW - 1)
        iv[pl.ds(j * L, L)] = p + rbase

    def _fire(t):
        return pltpu.async_copy(
            table_hbm.at[iv.at[pl.ds(t * CK, CK)]],
            rv.at[t % NBUF],
            sem,
        )

    cps = [_fire(t) for t in range(NBUF)]
    for t in range(NCH):
        cps[t % NBUF].wait()
        pltpu.sync_copy(rv.at[t % NBUF], out_hbm.at[pl.ds(kp0 + t * CK, CK)])
        if t + NBUF < NCH:
            cps[t % NBUF] = _fire(t + NBUF)


def kernel(original_kpts, segment):
    table = _transpose(segment).reshape(B * HW, CP)
    out = _gather_rows(original_kpts.reshape(-1), table)
    return out[:, :C].reshape(B, M, C)


# speedup vs baseline: 2.1275x; 1.2292x over previous
import dataclasses
import functools

import jax
import jax.numpy as jnp
from jax import lax
from jax.experimental import pallas as pl
from jax.experimental.pallas import tpu as pltpu
from jax.experimental.pallas import tpu_sc as plsc

B, M, C, H, W = 8, 2048, 192, 224, 224
HW = H * W
CP = 256

NC, NS, L = 2, 16, 16
NW = NC * NS
KPW = (B * M) // NW
CK = 128
NCH = KPW // CK
NBUF = 2

HB = 32

_mesh = plsc.VectorSubcoreMesh(core_axis_name="c", subcore_axis_name="s")

_cp = pltpu.CompilerParams()
if "needs_layout_passes" in pltpu.CompilerParams.__dataclass_fields__:
    _cp = dataclasses.replace(_cp, needs_layout_passes=False)


def _transpose_kernel(seg_ref, out_ref):
    x = seg_ref[0]
    for h in range(HB):
        out_ref[0, h, :, :C] = x[:, h, :].T


_transpose = pl.pallas_call(
    _transpose_kernel,
    grid=(B, H // HB),
    in_specs=[pl.BlockSpec((1, C, HB, W), lambda b, t: (b, 0, t, 0))],
    out_specs=pl.BlockSpec((1, HB, W, CP), lambda b, t: (b, t, 0, 0)),
    out_shape=jax.ShapeDtypeStruct((B, H, W, CP), jnp.float32),
)


@functools.partial(
    pl.kernel,
    mesh=_mesh,
    compiler_params=_cp,
    out_type=jax.ShapeDtypeStruct((B * M, CP), jnp.float32),
    scratch_types=[
        pltpu.VMEM((KPW * 2,), jnp.float32),
        pltpu.VMEM((KPW,), jnp.int32),
        pltpu.VMEM((NBUF, CK, CP), jnp.float32),
        pltpu.SemaphoreType.DMA,
    ],
)
def _gather_rows(kpts_hbm, table_hbm, out_hbm, kv, iv, rv, sem):
    wid = lax.axis_index("s") * NC + lax.axis_index("c")
    kp0 = wid * KPW
    b = kp0 // M
    rbase = b * HW

    lanes = lax.iota(jnp.int32, L)

    pltpu.sync_copy(kpts_hbm.at[pl.ds(kp0 * 2, KPW * 2)], kv)

    for j in range(KPW // L):
        ev = lanes * 2 + (j * 2 * L)
        xf = plsc.load_gather(kv, [ev]) * float(H)
        yf = plsc.load_gather(kv, [ev + 1]) * float(W)
        xi = xf.astype(jnp.int32)
        xi = jnp.where(xi.astype(jnp.float32) > xf, xi - 1, xi)
        yi = yf.astype(jnp.int32)
        yi = jnp.where(yi.astype(jnp.float32) > yf, yi - 1, yi)
        p = jnp.minimum(jnp.maximum(xi * W + yi, 0), HW - 1)
        iv[pl.ds(j * L, L)] = p + rbase

    def _fire(t):
        return pltpu.async_copy(
            table_hbm.at[iv.at[pl.ds(t * CK, CK)]],
            rv.at[t % NBUF],
            sem,
        )

    cps = [_fire(t) for t in range(NBUF)]
    for t in range(NCH):
        cps[t % NBUF].wait()
        pltpu.sync_copy(rv.at[t % NBUF], out_hbm.at[pl.ds(kp0 + t * CK, CK)])
        if t + NBUF < NCH:
            cps[t % NBUF] = _fire(t + NBUF)


def kernel(original_kpts, segment):
    table = _transpose(segment).reshape(B * HW, CP)
    out = _gather_rows(original_kpts.reshape(-1), table)
    return out[:, :C].reshape(B, M, C)
